# Initial kernel scaffold; baseline (speedup 1.0000x reference)
import jax
import jax.numpy as jnp
from jax.experimental import pallas as pl


def kernel(x, emb_table):
    raise NotImplementedError("write your pallas kernel here")


# speedup vs baseline: 20.2180x; 20.2180x over previous
import jax
import jax.numpy as jnp
from jax.experimental import pallas as pl
from jax.experimental.pallas import tpu as pltpu

S = 512
D = 128
T = 2 * S - 1
R = 16


def _body(rev_ref, x_ref, out_ref):
    xb = x_ref[:]
    base = pl.program_id(0) * R
    for r in range(R):
        start = (S - 1) - (base + r)
        out_ref[r] = xb + rev_ref[pl.ds(start, S), :]


def kernel(x, emb_table):
    rev = jnp.flip(emb_table, axis=0)
    xb = x[0]
    return pl.pallas_call(
        _body,
        grid=(S // R,),
        in_specs=[
            pl.BlockSpec((T, D), lambda i: (0, 0)),
            pl.BlockSpec((S, D), lambda i: (0, 0)),
        ],
        out_specs=pl.BlockSpec((R, S, D), lambda i: (i, 0, 0)),
        out_shape=jax.ShapeDtypeStruct((S, S, D), jnp.float32),
    )(rev, xb)
